# baseline (speedup 1.0000x reference)
import jax
import jax.numpy as jnp
from jax import lax
from jax.experimental import pallas as pl
from jax.experimental.pallas import tpu as pltpu
from jax.experimental.pallas import tpu_sc as plsc

_NUM_WORKERS = 32
_LANES = 16
_CHUNK = 4096


def _sc_gather_t(tables_t, cat_t, batch, row_lo, n_rows):
    nf, d, v = tables_t.shape
    rpw = n_rows // _NUM_WORKERS
    n_chunks = batch // _CHUNK
    assert n_chunks % 2 == 0
    mesh = plsc.VectorSubcoreMesh(core_axis_name="core", subcore_axis_name="subcore")

    @pl.kernel(
        out_type=jax.ShapeDtypeStruct((n_rows, batch), jnp.float32),
        mesh=mesh,
        compiler_params=pltpu.CompilerParams(needs_layout_passes=False),
        scratch_types=[
            pltpu.VMEM((1, v), jnp.float32),
            pltpu.VMEM((1, batch), jnp.int32),
            pltpu.VMEM((2, _CHUNK), jnp.float32),
            pltpu.SemaphoreType.DMA,
            pltpu.SemaphoreType.DMA,
            pltpu.SemaphoreType.DMA,
            pltpu.SemaphoreType.DMA,
        ],
    )
    def gather_kernel(tab_hbm, cat_hbm, out_hbm, row_v, idx_v, val_v, rsem, rsem2, s0, s1):
        wid = lax.axis_index("core") * 16 + lax.axis_index("subcore")
        zeros = jnp.zeros((_LANES,), jnp.int32)
        ssems = (s0, s1)

        @pl.loop(0, rpw)
        def _row(k):
            lr = wid * rpw + k
            r = row_lo + lr
            f = r // d
            j = r - f * d
            vh = (v // 2) // 128 * 128
            row_cp = pltpu.async_copy(
                tab_hbm.at[f, j, pl.ds(0, vh)], row_v.at[0, pl.ds(0, vh)], rsem
            )
            row_cp2 = pltpu.async_copy(
                tab_hbm.at[f, j, pl.ds(vh, v - vh)],
                row_v.at[0, pl.ds(vh, v - vh)],
                rsem2,
            )

            @pl.when(k > 0)
            def _():
                for s in ssems:
                    pltpu.make_async_copy(
                        val_v.at[0], out_hbm.at[0, pl.ds(0, _CHUNK)], s
                    ).wait()

            @pl.when((k == 0) | (f != (r - 1) // d))
            def _():
                pltpu.sync_copy(cat_hbm.at[f], idx_v.at[0])

            row_cp.wait()
            row_cp2.wait()

            for c in range(n_chunks):
                slot = c % 2
                base = c * _CHUNK
                if c >= 2:
                    pltpu.make_async_copy(
                        val_v.at[0], out_hbm.at[0, pl.ds(0, _CHUNK)], ssems[slot]
                    ).wait()

                @plsc.parallel_loop(0, _CHUNK, 16, unroll=8)
                def _blk(i):
                    idx = idx_v[0, pl.ds(base + i, _LANES)]
                    vals = plsc.load_gather(row_v, [zeros, idx])
                    val_v[slot, pl.ds(i, _LANES)] = vals

                pltpu.async_copy(
                    val_v.at[slot], out_hbm.at[lr, pl.ds(base, _CHUNK)], ssems[slot]
                )

        for s in ssems:
            pltpu.make_async_copy(
                val_v.at[0], out_hbm.at[0, pl.ds(0, _CHUNK)], s
            ).wait()

    return gather_kernel(tables_t, cat_t)


_BLK = 2048


def _tc_partial(w_t, emb_t, num_t, w1n_t):
    hidden, rows = w_t.shape
    batch = emb_t.shape[1]
    ndim = num_t.shape[0]

    def body(w_ref, e_ref, n_ref, wn_ref, o_ref):
        o_ref[...] = jax.lax.dot_general(
            w_ref[...], e_ref[...], (((1,), (0,)), ((), ())),
            precision=jax.lax.Precision.DEFAULT,
            preferred_element_type=jnp.float32,
        ) + jax.lax.dot_general(
            wn_ref[...], n_ref[...], (((1,), (0,)), ((), ())),
            precision=jax.lax.Precision.DEFAULT,
            preferred_element_type=jnp.float32,
        )

    return pl.pallas_call(
        body,
        grid=(batch // _BLK,),
        in_specs=[
            pl.BlockSpec((hidden, rows), lambda i: (0, 0)),
            pl.BlockSpec((rows, _BLK), lambda i: (0, i)),
            pl.BlockSpec((ndim, _BLK), lambda i: (0, i)),
            pl.BlockSpec((hidden, ndim), lambda i: (0, 0)),
        ],
        out_specs=pl.BlockSpec((hidden, _BLK), lambda i: (0, i)),
        out_shape=jax.ShapeDtypeStruct((hidden, batch), jnp.float32),
    )(w_t, emb_t, num_t, w1n_t)


def _tc_accum(partial, w_t, emb_t):
    hidden, rows = w_t.shape
    batch = emb_t.shape[1]

    def body(p_ref, w_ref, e_ref, o_ref):
        o_ref[...] = p_ref[...] + jax.lax.dot_general(
            w_ref[...], e_ref[...], (((1,), (0,)), ((), ())),
            precision=jax.lax.Precision.DEFAULT,
            preferred_element_type=jnp.float32,
        )

    return pl.pallas_call(
        body,
        grid=(batch // _BLK,),
        in_specs=[
            pl.BlockSpec((hidden, _BLK), lambda i: (0, i)),
            pl.BlockSpec((hidden, rows), lambda i: (0, 0)),
            pl.BlockSpec((rows, _BLK), lambda i: (0, i)),
        ],
        out_specs=pl.BlockSpec((hidden, _BLK), lambda i: (0, i)),
        out_shape=jax.ShapeDtypeStruct((hidden, batch), jnp.float32),
    )(partial, w_t, emb_t)


def _tc_final(partial, emb_t, w_t, b1_col, w2_col, b2):
    hidden, rows = w_t.shape
    batch = emb_t.shape[1]

    def body(p_ref, e_ref, w_ref, b1_ref, w2_ref, b2_ref, out_ref):
        ht = p_ref[...] + jax.lax.dot_general(
            w_ref[...], e_ref[...], (((1,), (0,)), ((), ())),
            precision=jax.lax.Precision.DEFAULT,
            preferred_element_type=jnp.float32,
        )
        ht = jnp.maximum(ht + b1_ref[...], 0.0)
        o = jnp.sum(ht * w2_ref[...], axis=0, keepdims=True) + b2_ref[...]
        out_ref[...] = jax.nn.sigmoid(o)

    return pl.pallas_call(
        body,
        grid=(batch // _BLK,),
        in_specs=[
            pl.BlockSpec((hidden, _BLK), lambda i: (0, i)),
            pl.BlockSpec((rows, _BLK), lambda i: (0, i)),
            pl.BlockSpec((hidden, rows), lambda i: (0, 0)),
            pl.BlockSpec((hidden, 1), lambda i: (0, 0)),
            pl.BlockSpec((hidden, 1), lambda i: (0, 0)),
            pl.BlockSpec((1, 1), lambda i: (0, 0)),
        ],
        out_specs=pl.BlockSpec((1, _BLK), lambda i: (0, i)),
        out_shape=jax.ShapeDtypeStruct((1, batch), jnp.float32),
    )(partial, emb_t, w_t, b1_col, w2_col, b2)


def kernel(categorical_inputs, numerical_inputs, tables, W1, b1, W2, b2):
    batch, nf = categorical_inputs.shape
    d = tables.shape[2]
    hidden = W1.shape[1]

    tables_t = jnp.transpose(tables, (0, 2, 1))
    cat_t = jnp.transpose(categorical_inputs, (1, 0))
    num_t = jnp.transpose(numerical_inputs, (1, 0))

    rows = nf * d
    field_splits = (13, 13) if nf == 26 else (nf,)

    w1e_t = jnp.transpose(W1[:rows], (1, 0))
    w1n_t = jnp.transpose(W1[rows:], (1, 0))

    bounds = []
    lo = 0
    for nfs in field_splits:
        bounds.append((lo * d, nfs * d))
        lo += nfs
    embs = [_sc_gather_t(tables_t, cat_t, batch, lo_r, n_r) for lo_r, n_r in bounds]

    partial = _tc_partial(w1e_t[:, : bounds[0][1]], embs[0], num_t, w1n_t)
    for (lo_r, n_r), emb in zip(bounds[1:-1], embs[1:-1]):
        partial = _tc_accum(partial, w1e_t[:, lo_r : lo_r + n_r], emb)
    lo_r, n_r = bounds[-1]
    out_row = _tc_final(
        partial,
        embs[-1],
        w1e_t[:, lo_r : lo_r + n_r],
        b1.reshape(hidden, 1),
        W2.reshape(hidden, 1),
        b2.reshape(1, 1),
    )
    return out_row.reshape(batch, 1)


# speedup vs baseline: 1.0209x; 1.0209x over previous
import jax
import jax.numpy as jnp
from jax import lax
from jax.experimental import pallas as pl
from jax.experimental.pallas import tpu as pltpu
from jax.experimental.pallas import tpu_sc as plsc

_NUM_WORKERS = 32
_LANES = 16
_CHUNK = 4096


def _sc_gather_t(tables_t, cat_t, batch, row_lo, n_rows):
    nf, d, v = tables_t.shape
    rpw = n_rows // _NUM_WORKERS
    n_chunks = batch // _CHUNK
    assert n_chunks % 2 == 0
    mesh = plsc.VectorSubcoreMesh(core_axis_name="core", subcore_axis_name="subcore")

    @pl.kernel(
        out_type=jax.ShapeDtypeStruct((n_rows, batch), jnp.float32),
        mesh=mesh,
        compiler_params=pltpu.CompilerParams(needs_layout_passes=False),
        scratch_types=[
            pltpu.VMEM((1, v), jnp.float32),
            pltpu.VMEM((1, batch), jnp.int32),
            pltpu.VMEM((2, _CHUNK), jnp.float32),
            pltpu.SemaphoreType.DMA,
            pltpu.SemaphoreType.DMA,
            pltpu.SemaphoreType.DMA,
        ],
    )
    def gather_kernel(tab_hbm, cat_hbm, out_hbm, row_v, idx_v, val_v, rsem, s0, s1):
        wid = lax.axis_index("core") * 16 + lax.axis_index("subcore")
        zeros = jnp.zeros((_LANES,), jnp.int32)
        ssems = (s0, s1)

        @pl.loop(0, rpw)
        def _row(k):
            lr = wid * rpw + k
            r = row_lo + lr
            f = r // d
            j = r - f * d
            row_cp = pltpu.async_copy(tab_hbm.at[f, j], row_v.at[0], rsem)

            @pl.when(k > 0)
            def _():
                for s in ssems:
                    pltpu.make_async_copy(
                        val_v.at[0], out_hbm.at[0, pl.ds(0, _CHUNK)], s
                    ).wait()

            @pl.when((k == 0) | (f != (r - 1) // d))
            def _():
                pltpu.sync_copy(cat_hbm.at[f], idx_v.at[0])

            row_cp.wait()

            for c in range(n_chunks):
                slot = c % 2
                base = c * _CHUNK
                if c >= 2:
                    pltpu.make_async_copy(
                        val_v.at[0], out_hbm.at[0, pl.ds(0, _CHUNK)], ssems[slot]
                    ).wait()

                @plsc.parallel_loop(0, _CHUNK, 16, unroll=16)
                def _blk(i):
                    idx = idx_v[0, pl.ds(base + i, _LANES)]
                    vals = plsc.load_gather(row_v, [zeros, idx])
                    val_v[slot, pl.ds(i, _LANES)] = vals

                pltpu.async_copy(
                    val_v.at[slot], out_hbm.at[lr, pl.ds(base, _CHUNK)], ssems[slot]
                )

        for s in ssems:
            pltpu.make_async_copy(
                val_v.at[0], out_hbm.at[0, pl.ds(0, _CHUNK)], s
            ).wait()

    return gather_kernel(tables_t, cat_t)


_BLK = 2048


def _tc_partial(w_t, emb_t, num_t, w1n_t):
    hidden, rows = w_t.shape
    batch = emb_t.shape[1]
    ndim = num_t.shape[0]

    def body(w_ref, e_ref, n_ref, wn_ref, o_ref):
        o_ref[...] = jax.lax.dot_general(
            w_ref[...], e_ref[...], (((1,), (0,)), ((), ())),
            precision=jax.lax.Precision.DEFAULT,
            preferred_element_type=jnp.float32,
        ) + jax.lax.dot_general(
            wn_ref[...], n_ref[...], (((1,), (0,)), ((), ())),
            precision=jax.lax.Precision.DEFAULT,
            preferred_element_type=jnp.float32,
        )

    return pl.pallas_call(
        body,
        grid=(batch // _BLK,),
        in_specs=[
            pl.BlockSpec((hidden, rows), lambda i: (0, 0)),
            pl.BlockSpec((rows, _BLK), lambda i: (0, i)),
            pl.BlockSpec((ndim, _BLK), lambda i: (0, i)),
            pl.BlockSpec((hidden, ndim), lambda i: (0, 0)),
        ],
        out_specs=pl.BlockSpec((hidden, _BLK), lambda i: (0, i)),
        out_shape=jax.ShapeDtypeStruct((hidden, batch), jnp.float32),
    )(w_t, emb_t, num_t, w1n_t)


def _tc_accum(partial, w_t, emb_t):
    hidden, rows = w_t.shape
    batch = emb_t.shape[1]

    def body(p_ref, w_ref, e_ref, o_ref):
        o_ref[...] = p_ref[...] + jax.lax.dot_general(
            w_ref[...], e_ref[...], (((1,), (0,)), ((), ())),
            precision=jax.lax.Precision.DEFAULT,
            preferred_element_type=jnp.float32,
        )

    return pl.pallas_call(
        body,
        grid=(batch // _BLK,),
        in_specs=[
            pl.BlockSpec((hidden, _BLK), lambda i: (0, i)),
            pl.BlockSpec((hidden, rows), lambda i: (0, 0)),
            pl.BlockSpec((rows, _BLK), lambda i: (0, i)),
        ],
        out_specs=pl.BlockSpec((hidden, _BLK), lambda i: (0, i)),
        out_shape=jax.ShapeDtypeStruct((hidden, batch), jnp.float32),
    )(partial, w_t, emb_t)


def _tc_final(partial, emb_t, w_t, b1_col, w2_col, b2):
    hidden, rows = w_t.shape
    batch = emb_t.shape[1]

    def body(p_ref, e_ref, w_ref, b1_ref, w2_ref, b2_ref, out_ref):
        ht = p_ref[...] + jax.lax.dot_general(
            w_ref[...], e_ref[...], (((1,), (0,)), ((), ())),
            precision=jax.lax.Precision.DEFAULT,
            preferred_element_type=jnp.float32,
        )
        ht = jnp.maximum(ht + b1_ref[...], 0.0)
        o = jnp.sum(ht * w2_ref[...], axis=0, keepdims=True) + b2_ref[...]
        out_ref[...] = jax.nn.sigmoid(o)

    return pl.pallas_call(
        body,
        grid=(batch // _BLK,),
        in_specs=[
            pl.BlockSpec((hidden, _BLK), lambda i: (0, i)),
            pl.BlockSpec((rows, _BLK), lambda i: (0, i)),
            pl.BlockSpec((hidden, rows), lambda i: (0, 0)),
            pl.BlockSpec((hidden, 1), lambda i: (0, 0)),
            pl.BlockSpec((hidden, 1), lambda i: (0, 0)),
            pl.BlockSpec((1, 1), lambda i: (0, 0)),
        ],
        out_specs=pl.BlockSpec((1, _BLK), lambda i: (0, i)),
        out_shape=jax.ShapeDtypeStruct((1, batch), jnp.float32),
    )(partial, emb_t, w_t, b1_col, w2_col, b2)


def kernel(categorical_inputs, numerical_inputs, tables, W1, b1, W2, b2):
    batch, nf = categorical_inputs.shape
    d = tables.shape[2]
    hidden = W1.shape[1]

    tables_t = jnp.transpose(tables, (0, 2, 1))
    cat_t = jnp.transpose(categorical_inputs, (1, 0))
    num_t = jnp.transpose(numerical_inputs, (1, 0))

    rows = nf * d
    field_splits = (13, 13) if nf == 26 else (nf,)

    w1e_t = jnp.transpose(W1[:rows], (1, 0))
    w1n_t = jnp.transpose(W1[rows:], (1, 0))

    bounds = []
    lo = 0
    for nfs in field_splits:
        bounds.append((lo * d, nfs * d))
        lo += nfs
    embs = [_sc_gather_t(tables_t, cat_t, batch, lo_r, n_r) for lo_r, n_r in bounds]

    partial = _tc_partial(w1e_t[:, : bounds[0][1]], embs[0], num_t, w1n_t)
    for (lo_r, n_r), emb in zip(bounds[1:-1], embs[1:-1]):
        partial = _tc_accum(partial, w1e_t[:, lo_r : lo_r + n_r], emb)
    lo_r, n_r = bounds[-1]
    out_row = _tc_final(
        partial,
        embs[-1],
        w1e_t[:, lo_r : lo_r + n_r],
        b1.reshape(hidden, 1),
        W2.reshape(hidden, 1),
        b2.reshape(1, 1),
    )
    return out_row.reshape(batch, 1)


# speedup vs baseline: 1.0270x; 1.0060x over previous
import jax
import jax.numpy as jnp
from jax import lax
from jax.experimental import pallas as pl
from jax.experimental.pallas import tpu as pltpu
from jax.experimental.pallas import tpu_sc as plsc

_NUM_WORKERS = 32
_LANES = 16
_CHUNK = 4096


def _sc_gather_t(tables_t, cat_t, batch, row_lo, n_rows):
    nf, d, v = tables_t.shape
    rpw = n_rows // _NUM_WORKERS
    n_chunks = batch // _CHUNK
    assert n_chunks % 2 == 0
    mesh = plsc.VectorSubcoreMesh(core_axis_name="core", subcore_axis_name="subcore")

    @pl.kernel(
        out_type=jax.ShapeDtypeStruct((n_rows, batch), jnp.float32),
        mesh=mesh,
        compiler_params=pltpu.CompilerParams(needs_layout_passes=False),
        scratch_types=[
            pltpu.VMEM((1, v), jnp.float32),
            pltpu.VMEM((1, batch), jnp.int32),
            pltpu.VMEM((2, _CHUNK), jnp.float32),
            pltpu.SemaphoreType.DMA,
            pltpu.SemaphoreType.DMA,
            pltpu.SemaphoreType.DMA,
        ],
    )
    def gather_kernel(tab_hbm, cat_hbm, out_hbm, row_v, idx_v, val_v, rsem, s0, s1):
        wid = lax.axis_index("core") * 16 + lax.axis_index("subcore")
        zeros = jnp.zeros((_LANES,), jnp.int32)
        ssems = (s0, s1)

        @pl.loop(0, rpw)
        def _row(k):
            lr = wid * rpw + k
            r = row_lo + lr
            f = r // d
            j = r - f * d
            row_cp = pltpu.async_copy(tab_hbm.at[f, j], row_v.at[0], rsem)

            @pl.when(k > 0)
            def _():
                for s in ssems:
                    pltpu.make_async_copy(
                        val_v.at[0], out_hbm.at[0, pl.ds(0, _CHUNK)], s
                    ).wait()

            @pl.when((k == 0) | (f != (r - 1) // d))
            def _():
                pltpu.sync_copy(cat_hbm.at[f], idx_v.at[0])

            row_cp.wait()

            for c in range(n_chunks):
                slot = c % 2
                base = c * _CHUNK
                if c >= 2:
                    pltpu.make_async_copy(
                        val_v.at[0], out_hbm.at[0, pl.ds(0, _CHUNK)], ssems[slot]
                    ).wait()

                @plsc.parallel_loop(0, _CHUNK, 16, unroll=16)
                def _blk(i):
                    idx = idx_v[0, pl.ds(base + i, _LANES)]
                    vals = plsc.load_gather(row_v, [zeros, idx])
                    val_v[slot, pl.ds(i, _LANES)] = vals

                pltpu.async_copy(
                    val_v.at[slot], out_hbm.at[lr, pl.ds(base, _CHUNK)], ssems[slot]
                )

        for s in ssems:
            pltpu.make_async_copy(
                val_v.at[0], out_hbm.at[0, pl.ds(0, _CHUNK)], s
            ).wait()

    return gather_kernel(tables_t, cat_t)


_BLK = 4096


def _tc_partial(w_t, emb_t, num_t, w1n_t):
    hidden, rows = w_t.shape
    batch = emb_t.shape[1]
    ndim = num_t.shape[0]

    def body(w_ref, e_ref, n_ref, wn_ref, o_ref):
        o_ref[...] = jax.lax.dot_general(
            w_ref[...], e_ref[...], (((1,), (0,)), ((), ())),
            precision=jax.lax.Precision.DEFAULT,
            preferred_element_type=jnp.float32,
        ) + jax.lax.dot_general(
            wn_ref[...], n_ref[...], (((1,), (0,)), ((), ())),
            precision=jax.lax.Precision.DEFAULT,
            preferred_element_type=jnp.float32,
        )

    return pl.pallas_call(
        body,
        grid=(batch // _BLK,),
        in_specs=[
            pl.BlockSpec((hidden, rows), lambda i: (0, 0)),
            pl.BlockSpec((rows, _BLK), lambda i: (0, i)),
            pl.BlockSpec((ndim, _BLK), lambda i: (0, i)),
            pl.BlockSpec((hidden, ndim), lambda i: (0, 0)),
        ],
        out_specs=pl.BlockSpec((hidden, _BLK), lambda i: (0, i)),
        out_shape=jax.ShapeDtypeStruct((hidden, batch), jnp.float32),
    )(w_t, emb_t, num_t, w1n_t)


def _tc_accum(partial, w_t, emb_t):
    hidden, rows = w_t.shape
    batch = emb_t.shape[1]

    def body(p_ref, w_ref, e_ref, o_ref):
        o_ref[...] = p_ref[...] + jax.lax.dot_general(
            w_ref[...], e_ref[...], (((1,), (0,)), ((), ())),
            precision=jax.lax.Precision.DEFAULT,
            preferred_element_type=jnp.float32,
        )

    return pl.pallas_call(
        body,
        grid=(batch // _BLK,),
        in_specs=[
            pl.BlockSpec((hidden, _BLK), lambda i: (0, i)),
            pl.BlockSpec((hidden, rows), lambda i: (0, 0)),
            pl.BlockSpec((rows, _BLK), lambda i: (0, i)),
        ],
        out_specs=pl.BlockSpec((hidden, _BLK), lambda i: (0, i)),
        out_shape=jax.ShapeDtypeStruct((hidden, batch), jnp.float32),
    )(partial, w_t, emb_t)


def _tc_final(partial, emb_t, w_t, b1_col, w2_col, b2):
    hidden, rows = w_t.shape
    batch = emb_t.shape[1]

    def body(p_ref, e_ref, w_ref, b1_ref, w2_ref, b2_ref, out_ref):
        ht = p_ref[...] + jax.lax.dot_general(
            w_ref[...], e_ref[...], (((1,), (0,)), ((), ())),
            precision=jax.lax.Precision.DEFAULT,
            preferred_element_type=jnp.float32,
        )
        ht = jnp.maximum(ht + b1_ref[...], 0.0)
        o = jnp.sum(ht * w2_ref[...], axis=0, keepdims=True) + b2_ref[...]
        out_ref[...] = jax.nn.sigmoid(o)

    return pl.pallas_call(
        body,
        grid=(batch // _BLK,),
        in_specs=[
            pl.BlockSpec((hidden, _BLK), lambda i: (0, i)),
            pl.BlockSpec((rows, _BLK), lambda i: (0, i)),
            pl.BlockSpec((hidden, rows), lambda i: (0, 0)),
            pl.BlockSpec((hidden, 1), lambda i: (0, 0)),
            pl.BlockSpec((hidden, 1), lambda i: (0, 0)),
            pl.BlockSpec((1, 1), lambda i: (0, 0)),
        ],
        out_specs=pl.BlockSpec((1, _BLK), lambda i: (0, i)),
        out_shape=jax.ShapeDtypeStruct((1, batch), jnp.float32),
    )(partial, emb_t, w_t, b1_col, w2_col, b2)


def kernel(categorical_inputs, numerical_inputs, tables, W1, b1, W2, b2):
    batch, nf = categorical_inputs.shape
    d = tables.shape[2]
    hidden = W1.shape[1]

    tables_t = jnp.transpose(tables, (0, 2, 1))
    cat_t = jnp.transpose(categorical_inputs, (1, 0))
    num_t = jnp.transpose(numerical_inputs, (1, 0))

    rows = nf * d
    field_splits = (13, 13) if nf == 26 else (nf,)

    w1e_t = jnp.transpose(W1[:rows], (1, 0))
    w1n_t = jnp.transpose(W1[rows:], (1, 0))

    bounds = []
    lo = 0
    for nfs in field_splits:
        bounds.append((lo * d, nfs * d))
        lo += nfs
    embs = [_sc_gather_t(tables_t, cat_t, batch, lo_r, n_r) for lo_r, n_r in bounds]

    partial = _tc_partial(w1e_t[:, : bounds[0][1]], embs[0], num_t, w1n_t)
    for (lo_r, n_r), emb in zip(bounds[1:-1], embs[1:-1]):
        partial = _tc_accum(partial, w1e_t[:, lo_r : lo_r + n_r], emb)
    lo_r, n_r = bounds[-1]
    out_row = _tc_final(
        partial,
        embs[-1],
        w1e_t[:, lo_r : lo_r + n_r],
        b1.reshape(hidden, 1),
        W2.reshape(hidden, 1),
        b2.reshape(1, 1),
    )
    return out_row.reshape(batch, 1)
